# Initial kernel scaffold; baseline (speedup 1.0000x reference)
import jax
import jax.numpy as jnp
from jax.experimental import pallas as pl


def kernel(features, coord, mask, lin_w0, lin_w1, lin_w2, mlp_w1, mlp_b1, mlp_w2, mlp_b2):
    raise NotImplementedError("write your pallas kernel here")


# speedup vs baseline: 21.1051x; 21.1051x over previous
import numpy as np
import jax
import jax.numpy as jnp
from jax.experimental import pallas as pl
from jax.experimental.pallas import tpu as pltpu

N = 4096
D = 128
K = 17
BINS = 32
M0, M1, M2 = 64, 16, 8
RB = 256
NB = N // RB
PAD = 8
STEP = np.float32(4.0 / (BINS - 1))
INV_STEP = np.float32((BINS - 1) / 4.0)
S3 = np.float32(np.sqrt(3.0))
S15 = np.float32(np.sqrt(15.0))
HS15 = np.float32(np.sqrt(15.0) / 2.0)
HS5 = np.float32(np.sqrt(5.0) / 2.0)
INV112 = np.float32(1.0 / 1.12)
INVK = np.float32(1.0 / K)
BIG = np.float32(1e30)

_VALS = (np.linspace(0.0, 4.0, BINS, dtype=np.float32)).reshape(1, BINS)

def _expander(m, c):
    e = np.zeros((m, m * c), np.float32)
    for o in range(m):
        e[o, o * c:(o + 1) * c] = 1.0
    return e

def _tiler(c, m):
    e = np.zeros((c, m * c), np.float32)
    for o in range(m):
        for j in range(c):
            e[j, o * c + j] = 1.0
    return e

_R16 = jnp.asarray(_expander(M1, 3))
_S3M = jnp.asarray(_tiler(3, M1))
_R8 = jnp.asarray(_expander(M2, 5))
_S5M = jnp.asarray(_tiler(5, M2))
_VALSJ = jnp.asarray(_VALS)


def _silu(x):
    return x * (1.0 / (1.0 + jnp.exp(-x)))


def _edge_math(u0cat, uccat, radcat, t0cat, t1cat, t2cat, sh1cat, sh2cat,
               b_ref, b1_ref, w2_ref, b2_ref, r16_ref, s3_ref, r8_ref, s5_ref,
               nrows):
    pre = (u0cat + uccat + b1_ref[...]
           + jnp.dot(radcat, b_ref[...], preferred_element_type=jnp.float32))
    h = _silu(pre)
    mix = jnp.dot(h, w2_ref[...], preferred_element_type=jnp.float32) + b2_ref[...]
    e0 = t0cat * mix[:, :M0]
    t1m = t1cat * mix[:, M0:M0 + M1]
    e1 = (jnp.dot(t1m, r16_ref[...], preferred_element_type=jnp.float32)
          * jnp.dot(sh1cat, s3_ref[...], preferred_element_type=jnp.float32))
    t2m = t2cat * mix[:, M0 + M1:]
    e2 = (jnp.dot(t2m, r8_ref[...], preferred_element_type=jnp.float32)
          * jnp.dot(sh2cat, s5_ref[...], preferred_element_type=jnp.float32))
    o0 = jnp.sum(e0.reshape(K, nrows, M0), axis=0) * INVK
    o1 = jnp.sum(e1.reshape(K, nrows, 3 * M1), axis=0) * INVK
    o2 = jnp.sum(e2.reshape(K, nrows, 5 * M2), axis=0) * INVK
    return jnp.concatenate([o0, o1, o2], axis=1)


def _geom(vec):
    x = vec[:, 0:1]
    y = vec[:, 1:2]
    z = vec[:, 2:3]
    ns = x * x + y * y + z * z
    norm = jnp.sqrt(jnp.where(ns == 0.0, 1.0, ns))
    vals = jax.lax.broadcasted_iota(jnp.int32, (1, BINS), 1).astype(jnp.float32) * STEP
    dd = (norm - vals) * INV_STEP
    rad = jnp.exp(-(dd * dd)) * INV112
    sh1 = S3 * vec
    sh2 = jnp.concatenate([
        S15 * (x * y), S15 * (y * z), HS5 * (2.0 * z * z - x * x - y * y),
        S15 * (x * z), HS15 * (x * x - y * y)], axis=1)
    return rad, sh1, sh2


def _pre_kernel(f_ref, w0_ref, w1_ref, w2l_ref, a_ref, c_ref,
                t0_ref, t1_ref, t2_ref, u0_ref, uc_ref):
    f = f_ref[...]
    t0 = jnp.dot(f, w0_ref[:D, :], preferred_element_type=jnp.float32) + w0_ref[D:D + 1, :]
    t1 = jnp.dot(f, w1_ref[:D, :], preferred_element_type=jnp.float32) + w1_ref[D:D + 1, :]
    t2 = jnp.dot(f, w2l_ref[:D, :], preferred_element_type=jnp.float32) + w2l_ref[D:D + 1, :]
    t0_ref[...] = t0
    t1_ref[...] = t1
    t2_ref[...] = t2
    u0_ref[...] = jnp.dot(t0, a_ref[...], preferred_element_type=jnp.float32)
    uc_ref[...] = jnp.dot(f, c_ref[...], preferred_element_type=jnp.float32)


def _band_kernel(t0_ref, t1_ref, t2_ref, u0_ref, uc_ref, co_ref,
                 b_ref, b1_ref, w2_ref, b2_ref, r16_ref, s3_ref, r8_ref, s5_ref,
                 out_ref):
    r0 = pl.program_id(0) * RB
    t0w = t0_ref[pl.ds(r0, RB + 2 * PAD), :]
    t1w = t1_ref[pl.ds(r0, RB + 2 * PAD), :]
    t2w = t2_ref[pl.ds(r0, RB + 2 * PAD), :]
    u0w = u0_ref[pl.ds(r0, RB + 2 * PAD), :]
    cow = co_ref[pl.ds(r0, RB + 2 * PAD), :]
    ucb = uc_ref[pl.ds(r0, RB), :]
    cc = cow[PAD:PAD + RB, :]
    rads, sh1s, sh2s, t0s, t1s, t2s, u0s = [], [], [], [], [], [], []
    for t in range(K):
        vec = cow[t:t + RB, :] - cc
        rad, sh1, sh2 = _geom(vec)
        rads.append(rad)
        sh1s.append(sh1)
        sh2s.append(sh2)
        t0s.append(t0w[t:t + RB, :])
        t1s.append(t1w[t:t + RB, :])
        t2s.append(t2w[t:t + RB, :])
        u0s.append(u0w[t:t + RB, :])
    out_ref[...] = _edge_math(
        jnp.concatenate(u0s, 0), jnp.concatenate([ucb] * K, 0),
        jnp.concatenate(rads, 0), jnp.concatenate(t0s, 0),
        jnp.concatenate(t1s, 0), jnp.concatenate(t2s, 0),
        jnp.concatenate(sh1s, 0), jnp.concatenate(sh2s, 0),
        b_ref, b1_ref, w2_ref, b2_ref, r16_ref, s3_ref, r8_ref, s5_ref, RB)


def _boundary_kernel(cot_ref, co_ref, t0_ref, t1_ref, t2_ref, u0_ref, uc_ref,
                     b_ref, b1_ref, w2_ref, b2_ref, r16_ref, s3_ref, r8_ref, s5_ref,
                     out_ref):
    bco = jnp.concatenate([co_ref[0:PAD, :], co_ref[N - PAD:N, :]], axis=0)
    d2 = jnp.zeros((2 * PAD, N), jnp.float32)
    for c in range(3):
        diff = cot_ref[c:c + 1, :] - bco[:, c:c + 1]
        d2 = d2 + diff * diff
    j2 = jax.lax.broadcasted_iota(jnp.int32, (2 * PAD, N), 1)
    r1 = jax.lax.broadcasted_iota(jnp.int32, (2 * PAD, 1), 0)
    low = jnp.where(r1 < PAD, r1 + PAD, -1)
    high = jnp.where(r1 < PAD, N + 1, (N - 24) + r1)
    d2m = jnp.where((j2 <= low) | (j2 >= high), BIG, d2)
    spats = []
    for _ in range(PAD):
        m = jnp.min(d2m, axis=1, keepdims=True)
        am = jnp.min(jnp.where(d2m == m, j2, N), axis=1, keepdims=True)
        spats.append(am)
        d2m = jnp.where(j2 == am, BIG, d2m)
    spat = jnp.concatenate(spats, axis=1)
    tt = jax.lax.broadcasted_iota(jnp.int32, (2 * PAD, K), 1)
    rr = jax.lax.broadcasted_iota(jnp.int32, (2 * PAD, K), 0)
    ig = jnp.where(rr < PAD, rr, (N - 2 * PAD) + rr)
    fixedcnt = jnp.where(rr < PAD, rr + 9, 24 - rr)
    base = jnp.where(rr < PAD, tt, ig - PAD + tt)
    s_idx = tt - fixedcnt
    gath = jnp.zeros((2 * PAD, K), jnp.int32)
    for s in range(PAD):
        gath = gath + jnp.where(s_idx == s, spat[:, s:s + 1], 0)
    nei = jnp.where(tt < fixedcnt, base, gath)
    ohs = []
    for t in range(K):
        ohs.append((j2 == nei[:, t:t + 1]).astype(jnp.float32))
    oh = jnp.concatenate(ohs, axis=0)
    g0 = jnp.dot(oh, t0_ref[...], preferred_element_type=jnp.float32)
    g1 = jnp.dot(oh, t1_ref[...], preferred_element_type=jnp.float32)
    g2 = jnp.dot(oh, t2_ref[...], preferred_element_type=jnp.float32)
    gu = jnp.dot(oh, u0_ref[...], preferred_element_type=jnp.float32)
    gco = jnp.dot(oh, co_ref[...], preferred_element_type=jnp.float32)
    ucb = jnp.concatenate([uc_ref[0:PAD, :], uc_ref[N - PAD:N, :]], axis=0)
    vec = gco - jnp.concatenate([bco] * K, 0)
    rad, sh1, sh2 = _geom(vec)
    out_ref[...] = _edge_math(
        gu, jnp.concatenate([ucb] * K, 0), rad, g0, g1, g2, sh1, sh2,
        b_ref, b1_ref, w2_ref, b2_ref, r16_ref, s3_ref, r8_ref, s5_ref, 2 * PAD)


def kernel(features, coord, mask, lin_w0, lin_w1, lin_w2, mlp_w1, mlp_b1, mlp_w2, mlp_b2):
    f32 = jnp.float32
    features = features.astype(f32)
    coord = coord.astype(f32)
    a_w = mlp_w1[:M0, :]
    b_w = mlp_w1[M0:M0 + BINS, :]
    c_w = mlp_w1[M0 + BINS:, :]
    b1 = mlp_b1.reshape(1, BINS)
    b2 = mlp_b2.reshape(1, M0 + M1 + M2)

    grid1 = 4
    rb1 = N // grid1
    t0f, t1f, t2f, u0f, ucf = pl.pallas_call(
        _pre_kernel,
        grid=(grid1,),
        in_specs=[
            pl.BlockSpec((rb1, D), lambda i: (i, 0)),
            pl.BlockSpec((D + 1, M0), lambda i: (0, 0)),
            pl.BlockSpec((D + 1, M1), lambda i: (0, 0)),
            pl.BlockSpec((D + 1, M2), lambda i: (0, 0)),
            pl.BlockSpec((M0, BINS), lambda i: (0, 0)),
            pl.BlockSpec((D, BINS), lambda i: (0, 0)),
        ],
        out_specs=[
            pl.BlockSpec((rb1, M0), lambda i: (i, 0)),
            pl.BlockSpec((rb1, M1), lambda i: (i, 0)),
            pl.BlockSpec((rb1, M2), lambda i: (i, 0)),
            pl.BlockSpec((rb1, BINS), lambda i: (i, 0)),
            pl.BlockSpec((rb1, BINS), lambda i: (i, 0)),
        ],
        out_shape=[
            jax.ShapeDtypeStruct((N, M0), f32),
            jax.ShapeDtypeStruct((N, M1), f32),
            jax.ShapeDtypeStruct((N, M2), f32),
            jax.ShapeDtypeStruct((N, BINS), f32),
            jax.ShapeDtypeStruct((N, BINS), f32),
        ],
    )(features, lin_w0, lin_w1, lin_w2, a_w, c_w)

    pad = ((PAD, PAD), (0, 0))
    t0p = jnp.pad(t0f, pad)
    t1p = jnp.pad(t1f, pad)
    t2p = jnp.pad(t2f, pad)
    u0p = jnp.pad(u0f, pad)
    cop = jnp.pad(coord, pad)

    npad = N + 2 * PAD
    wfull = lambda shp: pl.BlockSpec(shp, lambda i: tuple(0 for _ in shp))
    main = pl.pallas_call(
        _band_kernel,
        grid=(NB,),
        in_specs=[
            wfull((npad, M0)), wfull((npad, M1)), wfull((npad, M2)),
            wfull((npad, BINS)), wfull((N, BINS)), wfull((npad, 3)),
            wfull((BINS, BINS)), wfull((1, BINS)),
            wfull((BINS, M0 + M1 + M2)), wfull((1, M0 + M1 + M2)),
            wfull((M1, 3 * M1)), wfull((3, 3 * M1)),
            wfull((M2, 5 * M2)), wfull((5, 5 * M2)),
        ],
        out_specs=pl.BlockSpec((RB, M0 + 3 * M1 + 5 * M2), lambda i: (i, 0)),
        out_shape=jax.ShapeDtypeStruct((N, M0 + 3 * M1 + 5 * M2), f32),
    )(t0p, t1p, t2p, u0p, ucf, cop, b_w, b1, mlp_w2, b2, _R16, _S3M, _R8, _S5M)

    fix = pl.pallas_call(
        _boundary_kernel,
        out_shape=jax.ShapeDtypeStruct((2 * PAD, M0 + 3 * M1 + 5 * M2), f32),
    )(coord.T, coord, t0f, t1f, t2f, u0f, ucf,
      b_w, b1, mlp_w2, b2, _R16, _S3M, _R8, _S5M)

    return jnp.concatenate([fix[:PAD], main[PAD:N - PAD], fix[PAD:]], axis=0)


# speedup vs baseline: 58.0156x; 2.7489x over previous
import numpy as np
import jax
import jax.numpy as jnp
from jax.experimental import pallas as pl
from jax.experimental.pallas import tpu as pltpu

N = 4096
D = 128
K = 17
BINS = 32
M0, M1, M2 = 64, 16, 8
RB = 256
NB = N // RB
PAD = 8
NP2 = N + 2 * PAD
STEP = np.float32(4.0 / (BINS - 1))
INV_STEP = np.float32((BINS - 1) / 4.0)
S3 = np.float32(np.sqrt(3.0))
S15 = np.float32(np.sqrt(15.0))
HS15 = np.float32(np.sqrt(15.0) / 2.0)
HS5 = np.float32(np.sqrt(5.0) / 2.0)
INV112 = np.float32(1.0 / 1.12)
INVK = np.float32(1.0 / K)
BIG = np.float32(1e30)
MOUT = M0 + 3 * M1 + 5 * M2


def _expander(m, c):
    e = np.zeros((m, m * c), np.float32)
    for o in range(m):
        e[o, o * c:(o + 1) * c] = 1.0
    return e

def _tiler(c, m):
    e = np.zeros((c, m * c), np.float32)
    for o in range(m):
        for j in range(c):
            e[j, o * c + j] = 1.0
    return e

_R16 = _expander(M1, 3)
_S3M = _tiler(3, M1)
_R8 = _expander(M2, 5)
_S5M = _tiler(5, M2)


def _silu(x):
    return x * (1.0 / (1.0 + jnp.exp(-x)))


def _pre_kernel(f_ref, ft_ref, w0_ref, w1_ref, w2l_ref, a_ref, c_ref,
                w0t_ref, w1t_ref, w2lt_ref, at_ref, ct_ref,
                t0_ref, t1_ref, t2_ref, u0_ref, uc_ref,
                t0t_ref, t1t_ref, t2t_ref, u0t_ref, uct_ref):
    pid = pl.program_id(0)
    f = f_ref[...]
    t0 = jnp.dot(f, w0_ref[:D, :], preferred_element_type=jnp.float32) + w0_ref[D:D + 1, :]
    t1 = jnp.dot(f, w1_ref[:D, :], preferred_element_type=jnp.float32) + w1_ref[D:D + 1, :]
    t2 = jnp.dot(f, w2l_ref[:D, :], preferred_element_type=jnp.float32) + w2l_ref[D:D + 1, :]
    t0_ref[...] = t0
    t1_ref[...] = t1
    t2_ref[...] = t2
    u0_ref[...] = jnp.dot(t0, a_ref[...], preferred_element_type=jnp.float32)
    uc_ref[...] = jnp.dot(f, c_ref[...], preferred_element_type=jnp.float32)

    ft = ft_ref[...]
    t0t = jnp.dot(w0t_ref[:, :D], ft, preferred_element_type=jnp.float32) + w0t_ref[:, D:D + 1]
    t1t = jnp.dot(w1t_ref[:, :D], ft, preferred_element_type=jnp.float32) + w1t_ref[:, D:D + 1]
    t2t = jnp.dot(w2lt_ref[:, :D], ft, preferred_element_type=jnp.float32) + w2lt_ref[:, D:D + 1]
    t0t_ref[...] = t0t
    t1t_ref[...] = t1t
    t2t_ref[...] = t2t
    u0t_ref[...] = jnp.dot(at_ref[...], t0t, preferred_element_type=jnp.float32)
    uct_ref[...] = jnp.dot(ct_ref[...], ft, preferred_element_type=jnp.float32)


def _band_kernel(t0t_ref, t1t_ref, t2t_ref, u0t_ref, uct_ref, cot_ref,
                 bt_ref, b1c_ref, w2t_ref, b2c_ref,
                 r16t_ref, s3t_ref, r8t_ref, s5t_ref,
                 out_ref):
    r0 = pl.program_id(0) * RB
    w = RB + 2 * PAD
    cow = cot_ref[:, pl.ds(r0, w)]
    ccx = cow[0:1, PAD:PAD + RB]
    ccy = cow[1:2, PAD:PAD + RB]
    ccz = cow[2:3, PAD:PAD + RB]
    xs, ys, zs = [], [], []
    for t in range(K):
        xs.append(cow[0:1, t:t + RB] - ccx)
        ys.append(cow[1:2, t:t + RB] - ccy)
        zs.append(cow[2:3, t:t + RB] - ccz)
    x = jnp.concatenate(xs, 1)
    y = jnp.concatenate(ys, 1)
    z = jnp.concatenate(zs, 1)
    xx = x * x
    yy = y * y
    zz = z * z
    ns = xx + yy + zz
    norm = jnp.sqrt(jnp.where(ns == 0.0, 1.0, ns))
    valc = jax.lax.broadcasted_iota(jnp.int32, (BINS, 1), 0).astype(jnp.float32) * STEP
    dd = (norm - valc) * INV_STEP
    rad = jnp.exp(-(dd * dd)) * INV112
    sh1 = S3 * jnp.concatenate([x, y, z], 0)
    sh2 = jnp.concatenate([
        S15 * (x * y), S15 * (y * z), HS5 * (2.0 * zz - xx - yy),
        S15 * (x * z), HS15 * (xx - yy)], 0)

    t0w = t0t_ref[:, pl.ds(r0, w)]
    t1w = t1t_ref[:, pl.ds(r0, w)]
    t2w = t2t_ref[:, pl.ds(r0, w)]
    u0w = u0t_ref[:, pl.ds(r0, w)]
    t0c = jnp.concatenate([t0w[:, t:t + RB] for t in range(K)], 1)
    t1c = jnp.concatenate([t1w[:, t:t + RB] for t in range(K)], 1)
    t2c = jnp.concatenate([t2w[:, t:t + RB] for t in range(K)], 1)
    u0c = jnp.concatenate([u0w[:, t:t + RB] for t in range(K)], 1)
    ucb = uct_ref[:, pl.ds(r0, RB)]
    ucc = jnp.concatenate([ucb] * K, 1)

    pre = (u0c + ucc + b1c_ref[...]
           + jnp.dot(bt_ref[...], rad, preferred_element_type=jnp.float32))
    h = _silu(pre)
    mix = jnp.dot(w2t_ref[...], h, preferred_element_type=jnp.float32) + b2c_ref[...]
    e0 = t0c * mix[:M0, :]
    t1m = t1c * mix[M0:M0 + M1, :]
    e1 = (jnp.dot(r16t_ref[...], t1m, preferred_element_type=jnp.float32)
          * jnp.dot(s3t_ref[...], sh1, preferred_element_type=jnp.float32))
    t2m = t2c * mix[M0 + M1:, :]
    e2 = (jnp.dot(r8t_ref[...], t2m, preferred_element_type=jnp.float32)
          * jnp.dot(s5t_ref[...], sh2, preferred_element_type=jnp.float32))
    o0 = e0[:, 0:RB]
    o1 = e1[:, 0:RB]
    o2 = e2[:, 0:RB]
    for t in range(1, K):
        o0 = o0 + e0[:, t * RB:(t + 1) * RB]
        o1 = o1 + e1[:, t * RB:(t + 1) * RB]
        o2 = o2 + e2[:, t * RB:(t + 1) * RB]
    out_ref[...] = jnp.concatenate([o0, o1, o2], 0) * INVK


def _geom_rows(vec):
    x = vec[:, 0:1]
    y = vec[:, 1:2]
    z = vec[:, 2:3]
    ns = x * x + y * y + z * z
    norm = jnp.sqrt(jnp.where(ns == 0.0, 1.0, ns))
    vals = jax.lax.broadcasted_iota(jnp.int32, (1, BINS), 1).astype(jnp.float32) * STEP
    dd = (norm - vals) * INV_STEP
    rad = jnp.exp(-(dd * dd)) * INV112
    sh1 = S3 * vec
    sh2 = jnp.concatenate([
        S15 * (x * y), S15 * (y * z), HS5 * (2.0 * z * z - x * x - y * y),
        S15 * (x * z), HS15 * (x * x - y * y)], axis=1)
    return rad, sh1, sh2


def _boundary_kernel(cot_ref, co_ref, t0_ref, t1_ref, t2_ref, u0_ref, uc_ref,
                     b_ref, b1_ref, w2_ref, b2_ref, r16_ref, s3_ref, r8_ref, s5_ref,
                     out_ref):
    bco = jnp.concatenate([co_ref[0:PAD, :], co_ref[N - PAD:N, :]], axis=0)
    d2 = jnp.zeros((2 * PAD, N), jnp.float32)
    for c in range(3):
        diff = cot_ref[c:c + 1, :] - bco[:, c:c + 1]
        d2 = d2 + diff * diff
    j2 = jax.lax.broadcasted_iota(jnp.int32, (2 * PAD, N), 1)
    r1 = jax.lax.broadcasted_iota(jnp.int32, (2 * PAD, 1), 0)
    low = jnp.where(r1 < PAD, r1 + PAD, -1)
    high = jnp.where(r1 < PAD, N + 1, (N - 24) + r1)
    d2m = jnp.where((j2 <= low) | (j2 >= high), BIG, d2)
    spats = []
    for _ in range(PAD):
        m = jnp.min(d2m, axis=1, keepdims=True)
        am = jnp.min(jnp.where(d2m == m, j2, N), axis=1, keepdims=True)
        spats.append(am)
        d2m = jnp.where(j2 == am, BIG, d2m)
    spat = jnp.concatenate(spats, axis=1)
    tt = jax.lax.broadcasted_iota(jnp.int32, (2 * PAD, K), 1)
    rr = jax.lax.broadcasted_iota(jnp.int32, (2 * PAD, K), 0)
    ig = jnp.where(rr < PAD, rr, (N - 2 * PAD) + rr)
    fixedcnt = jnp.where(rr < PAD, rr + 9, 24 - rr)
    base = jnp.where(rr < PAD, tt, ig - PAD + tt)
    s_idx = tt - fixedcnt
    gath = jnp.zeros((2 * PAD, K), jnp.int32)
    for s in range(PAD):
        gath = gath + jnp.where(s_idx == s, spat[:, s:s + 1], 0)
    nei = jnp.where(tt < fixedcnt, base, gath)
    ohs = []
    for t in range(K):
        ohs.append((j2 == nei[:, t:t + 1]).astype(jnp.float32))
    oh = jnp.concatenate(ohs, axis=0)
    g0 = jnp.dot(oh, t0_ref[...], preferred_element_type=jnp.float32)
    g1 = jnp.dot(oh, t1_ref[...], preferred_element_type=jnp.float32)
    g2 = jnp.dot(oh, t2_ref[...], preferred_element_type=jnp.float32)
    gu = jnp.dot(oh, u0_ref[...], preferred_element_type=jnp.float32)
    gco = jnp.dot(oh, co_ref[...], preferred_element_type=jnp.float32)
    ucb = jnp.concatenate([uc_ref[0:PAD, :], uc_ref[N - PAD:N, :]], axis=0)
    vec = gco - jnp.concatenate([bco] * K, 0)
    rad, sh1, sh2 = _geom_rows(vec)
    pre = (gu + jnp.concatenate([ucb] * K, 0) + b1_ref[...]
           + jnp.dot(rad, b_ref[...], preferred_element_type=jnp.float32))
    h = _silu(pre)
    mix = jnp.dot(h, w2_ref[...], preferred_element_type=jnp.float32) + b2_ref[...]
    e0 = g0 * mix[:, :M0]
    t1m = g1 * mix[:, M0:M0 + M1]
    e1 = (jnp.dot(t1m, r16_ref[...], preferred_element_type=jnp.float32)
          * jnp.dot(sh1, s3_ref[...], preferred_element_type=jnp.float32))
    t2m = g2 * mix[:, M0 + M1:]
    e2 = (jnp.dot(t2m, r8_ref[...], preferred_element_type=jnp.float32)
          * jnp.dot(sh2, s5_ref[...], preferred_element_type=jnp.float32))
    o0 = jnp.sum(e0.reshape(K, 2 * PAD, M0), axis=0) * INVK
    o1 = jnp.sum(e1.reshape(K, 2 * PAD, 3 * M1), axis=0) * INVK
    o2 = jnp.sum(e2.reshape(K, 2 * PAD, 5 * M2), axis=0) * INVK
    out_ref[...] = jnp.concatenate([o0, o1, o2], axis=1)


def kernel(features, coord, mask, lin_w0, lin_w1, lin_w2, mlp_w1, mlp_b1, mlp_w2, mlp_b2):
    f32 = jnp.float32
    features = features.astype(f32)
    coord = coord.astype(f32)
    a_w = mlp_w1[:M0, :]
    b_w = mlp_w1[M0:M0 + BINS, :]
    c_w = mlp_w1[M0 + BINS:, :]
    b1 = mlp_b1.reshape(1, BINS)
    b2 = mlp_b2.reshape(1, M0 + M1 + M2)
    ft = features.T
    cot = coord.T
    cotp = jnp.pad(cot, ((0, 0), (PAD, PAD)))

    grid1 = 4
    rb1 = N // grid1
    wcol = lambda shp: pl.BlockSpec(shp, lambda i: (0, 0))
    (t0f, t1f, t2f, u0f, ucf, t0t, t1t, t2t, u0t, uct) = pl.pallas_call(
        _pre_kernel,
        grid=(grid1,),
        in_specs=[
            pl.BlockSpec((rb1, D), lambda i: (i, 0)),
            pl.BlockSpec((D, rb1), lambda i: (0, i)),
            wcol((D + 1, M0)), wcol((D + 1, M1)), wcol((D + 1, M2)),
            wcol((M0, BINS)), wcol((D, BINS)),
            wcol((M0, D + 1)), wcol((M1, D + 1)), wcol((M2, D + 1)),
            wcol((BINS, M0)), wcol((BINS, D)),
        ],
        out_specs=[
            pl.BlockSpec((rb1, M0), lambda i: (i, 0)),
            pl.BlockSpec((rb1, M1), lambda i: (i, 0)),
            pl.BlockSpec((rb1, M2), lambda i: (i, 0)),
            pl.BlockSpec((rb1, BINS), lambda i: (i, 0)),
            pl.BlockSpec((rb1, BINS), lambda i: (i, 0)),
            pl.BlockSpec((M0, rb1), lambda i: (0, i)),
            pl.BlockSpec((M1, rb1), lambda i: (0, i)),
            pl.BlockSpec((M2, rb1), lambda i: (0, i)),
            pl.BlockSpec((BINS, rb1), lambda i: (0, i)),
            pl.BlockSpec((BINS, rb1), lambda i: (0, i)),
        ],
        out_shape=[
            jax.ShapeDtypeStruct((N, M0), f32),
            jax.ShapeDtypeStruct((N, M1), f32),
            jax.ShapeDtypeStruct((N, M2), f32),
            jax.ShapeDtypeStruct((N, BINS), f32),
            jax.ShapeDtypeStruct((N, BINS), f32),
            jax.ShapeDtypeStruct((M0, N), f32),
            jax.ShapeDtypeStruct((M1, N), f32),
            jax.ShapeDtypeStruct((M2, N), f32),
            jax.ShapeDtypeStruct((BINS, N), f32),
            jax.ShapeDtypeStruct((BINS, N), f32),
        ],
    )(features, ft, lin_w0, lin_w1, lin_w2, a_w, c_w,
      lin_w0.T, lin_w1.T, lin_w2.T, a_w.T, c_w.T)
    lpad = ((0, 0), (PAD, PAD))
    t0t = jnp.pad(t0t, lpad)
    t1t = jnp.pad(t1t, lpad)
    t2t = jnp.pad(t2t, lpad)
    u0t = jnp.pad(u0t, lpad)

    maint = pl.pallas_call(
        _band_kernel,
        grid=(NB,),
        in_specs=[
            wcol((M0, NP2)), wcol((M1, NP2)), wcol((M2, NP2)),
            wcol((BINS, NP2)), wcol((BINS, N)), wcol((3, NP2)),
            wcol((BINS, BINS)), wcol((BINS, 1)),
            wcol((M0 + M1 + M2, BINS)), wcol((M0 + M1 + M2, 1)),
            wcol((3 * M1, M1)), wcol((3 * M1, 3)),
            wcol((5 * M2, M2)), wcol((5 * M2, 5)),
        ],
        out_specs=pl.BlockSpec((MOUT, RB), lambda i: (0, i)),
        out_shape=jax.ShapeDtypeStruct((MOUT, N), f32),
    )(t0t, t1t, t2t, u0t, uct, cotp,
      b_w.T, mlp_b1.reshape(BINS, 1), mlp_w2.T, mlp_b2.reshape(M0 + M1 + M2, 1),
      _R16.T, _S3M.T, _R8.T, _S5M.T)

    fix = pl.pallas_call(
        _boundary_kernel,
        out_shape=jax.ShapeDtypeStruct((2 * PAD, MOUT), f32),
    )(cot, coord, t0f, t1f, t2f, u0f, ucf,
      b_w, b1, mlp_w2, b2, _R16, _S3M, _R8, _S5M)

    return jnp.concatenate([fix[:PAD], maint.T[PAD:N - PAD], fix[PAD:]], axis=0)


# speedup vs baseline: 64.6794x; 1.1149x over previous
import numpy as np
import jax
import jax.numpy as jnp
from jax.experimental import pallas as pl
from jax.experimental.pallas import tpu as pltpu

N = 4096
D = 128
K = 17
BINS = 32
M0, M1, M2 = 64, 16, 8
RB = 256
NB = N // RB
PAD = 8
NP2 = N + 2 * PAD
STEP = np.float32(4.0 / (BINS - 1))
INV_STEP = np.float32((BINS - 1) / 4.0)
S3 = np.float32(np.sqrt(3.0))
S15 = np.float32(np.sqrt(15.0))
HS15 = np.float32(np.sqrt(15.0) / 2.0)
HS5 = np.float32(np.sqrt(5.0) / 2.0)
INV112 = np.float32(1.0 / 1.12)
INVK = np.float32(1.0 / K)
BIG = np.float32(1e30)
MOUT = M0 + 3 * M1 + 5 * M2


def _expander(m, c):
    e = np.zeros((m, m * c), np.float32)
    for o in range(m):
        e[o, o * c:(o + 1) * c] = 1.0
    return e

def _tiler(c, m):
    e = np.zeros((c, m * c), np.float32)
    for o in range(m):
        for j in range(c):
            e[j, o * c + j] = 1.0
    return e

_R16 = _expander(M1, 3)
_S3M = _tiler(3, M1)
_R8 = _expander(M2, 5)
_S5M = _tiler(5, M2)


def _silu(x):
    return x * (1.0 / (1.0 + jnp.exp(-x)))


def _dot(a, b):
    return jnp.dot(a, b, preferred_element_type=jnp.float32)


def _band_kernel(fp_ref, cop_ref,
                 w0t_ref, w1t_ref, w2lt_ref, at_ref, ct_ref,
                 bt_ref, b1c_ref, w2t_ref, b2c_ref,
                 r16t_ref, s3t_ref, r8t_ref, s5t_ref,
                 out_ref):
    r0 = pl.program_id(0) * RB
    w = RB + 2 * PAD
    ftw = fp_ref[pl.ds(r0, w), :].T
    cow = cop_ref[pl.ds(r0, w), :].T
    t0w = _dot(w0t_ref[:, :D], ftw) + w0t_ref[:, D:D + 1]
    t1w = _dot(w1t_ref[:, :D], ftw) + w1t_ref[:, D:D + 1]
    t2w = _dot(w2lt_ref[:, :D], ftw) + w2lt_ref[:, D:D + 1]
    u0w = _dot(at_ref[...], t0w)
    ucb = _dot(ct_ref[...], ftw[:, PAD:PAD + RB])

    ccx = cow[0:1, PAD:PAD + RB]
    ccy = cow[1:2, PAD:PAD + RB]
    ccz = cow[2:3, PAD:PAD + RB]
    xs, ys, zs = [], [], []
    for t in range(K):
        xs.append(cow[0:1, t:t + RB] - ccx)
        ys.append(cow[1:2, t:t + RB] - ccy)
        zs.append(cow[2:3, t:t + RB] - ccz)
    x = jnp.concatenate(xs, 1)
    y = jnp.concatenate(ys, 1)
    z = jnp.concatenate(zs, 1)
    xx = x * x
    yy = y * y
    zz = z * z
    ns = xx + yy + zz
    norm = jnp.sqrt(jnp.where(ns == 0.0, 1.0, ns))
    valc = jax.lax.broadcasted_iota(jnp.int32, (BINS, 1), 0).astype(jnp.float32) * STEP
    dd = (norm - valc) * INV_STEP
    rad = jnp.exp(-(dd * dd)) * INV112
    sh1 = S3 * jnp.concatenate([x, y, z], 0)
    sh2 = jnp.concatenate([
        S15 * (x * y), S15 * (y * z), HS5 * (2.0 * zz - xx - yy),
        S15 * (x * z), HS15 * (xx - yy)], 0)

    t0c = jnp.concatenate([t0w[:, t:t + RB] for t in range(K)], 1)
    t1c = jnp.concatenate([t1w[:, t:t + RB] for t in range(K)], 1)
    t2c = jnp.concatenate([t2w[:, t:t + RB] for t in range(K)], 1)
    u0c = jnp.concatenate([u0w[:, t:t + RB] for t in range(K)], 1)
    ucc = jnp.concatenate([ucb] * K, 1)

    pre = u0c + ucc + b1c_ref[...] + _dot(bt_ref[...], rad)
    h = _silu(pre)
    mix = _dot(w2t_ref[...], h) + b2c_ref[...]
    e0 = t0c * mix[:M0, :]
    t1m = t1c * mix[M0:M0 + M1, :]
    e1 = _dot(r16t_ref[...], t1m) * _dot(s3t_ref[...], sh1)
    t2m = t2c * mix[M0 + M1:, :]
    e2 = _dot(r8t_ref[...], t2m) * _dot(s5t_ref[...], sh2)
    o0 = e0[:, 0:RB]
    o1 = e1[:, 0:RB]
    o2 = e2[:, 0:RB]
    for t in range(1, K):
        o0 = o0 + e0[:, t * RB:(t + 1) * RB]
        o1 = o1 + e1[:, t * RB:(t + 1) * RB]
        o2 = o2 + e2[:, t * RB:(t + 1) * RB]
    out_ref[...] = (jnp.concatenate([o0, o1, o2], 0) * INVK).T


def _geom_rows(vec):
    x = vec[:, 0:1]
    y = vec[:, 1:2]
    z = vec[:, 2:3]
    ns = x * x + y * y + z * z
    norm = jnp.sqrt(jnp.where(ns == 0.0, 1.0, ns))
    vals = jax.lax.broadcasted_iota(jnp.int32, (1, BINS), 1).astype(jnp.float32) * STEP
    dd = (norm - vals) * INV_STEP
    rad = jnp.exp(-(dd * dd)) * INV112
    sh1 = S3 * vec
    sh2 = jnp.concatenate([
        S15 * (x * y), S15 * (y * z), HS5 * (2.0 * z * z - x * x - y * y),
        S15 * (x * z), HS15 * (x * x - y * y)], axis=1)
    return rad, sh1, sh2


def _boundary_kernel(cot_ref, co_ref, f_ref,
                     w0_ref, w1_ref, w2l_ref, a_ref, c_ref,
                     b_ref, b1_ref, w2_ref, b2_ref, r16_ref, s3_ref, r8_ref, s5_ref,
                     out_ref):
    bco = jnp.concatenate([co_ref[0:PAD, :], co_ref[N - PAD:N, :]], axis=0)
    d2 = jnp.zeros((2 * PAD, N), jnp.float32)
    for c in range(3):
        diff = cot_ref[c:c + 1, :] - bco[:, c:c + 1]
        d2 = d2 + diff * diff
    j2 = jax.lax.broadcasted_iota(jnp.int32, (2 * PAD, N), 1)
    r1 = jax.lax.broadcasted_iota(jnp.int32, (2 * PAD, 1), 0)
    low = jnp.where(r1 < PAD, r1 + PAD, -1)
    high = jnp.where(r1 < PAD, N + 1, (N - 24) + r1)
    d2m = jnp.where((j2 <= low) | (j2 >= high), BIG, d2)
    spats = []
    for _ in range(PAD):
        m = jnp.min(d2m, axis=1, keepdims=True)
        am = jnp.min(jnp.where(d2m == m, j2, N), axis=1, keepdims=True)
        spats.append(am)
        d2m = jnp.where(j2 == am, BIG, d2m)
    spat = jnp.concatenate(spats, axis=1)
    tt = jax.lax.broadcasted_iota(jnp.int32, (2 * PAD, K), 1)
    rr = jax.lax.broadcasted_iota(jnp.int32, (2 * PAD, K), 0)
    ig = jnp.where(rr < PAD, rr, (N - 2 * PAD) + rr)
    fixedcnt = jnp.where(rr < PAD, rr + 9, 24 - rr)
    base = jnp.where(rr < PAD, tt, ig - PAD + tt)
    s_idx = tt - fixedcnt
    gath = jnp.zeros((2 * PAD, K), jnp.int32)
    for s in range(PAD):
        gath = gath + jnp.where(s_idx == s, spat[:, s:s + 1], 0)
    nei = jnp.where(tt < fixedcnt, base, gath)
    ohs = []
    for t in range(K):
        ohs.append((j2 == nei[:, t:t + 1]).astype(jnp.float32))
    oh = jnp.concatenate(ohs, axis=0)
    gf = _dot(oh, f_ref[...])
    gco = _dot(oh, co_ref[...])
    g0 = _dot(gf, w0_ref[:D, :]) + w0_ref[D:D + 1, :]
    g1 = _dot(gf, w1_ref[:D, :]) + w1_ref[D:D + 1, :]
    g2 = _dot(gf, w2l_ref[:D, :]) + w2l_ref[D:D + 1, :]
    gu = _dot(g0, a_ref[...])
    fc = jnp.concatenate([f_ref[0:PAD, :], f_ref[N - PAD:N, :]], axis=0)
    ucb = _dot(fc, c_ref[...])
    vec = gco - jnp.concatenate([bco] * K, 0)
    rad, sh1, sh2 = _geom_rows(vec)
    pre = (gu + jnp.concatenate([ucb] * K, 0) + b1_ref[...] + _dot(rad, b_ref[...]))
    h = _silu(pre)
    mix = _dot(h, w2_ref[...]) + b2_ref[...]
    e0 = g0 * mix[:, :M0]
    t1m = g1 * mix[:, M0:M0 + M1]
    e1 = _dot(t1m, r16_ref[...]) * _dot(sh1, s3_ref[...])
    t2m = g2 * mix[:, M0 + M1:]
    e2 = _dot(t2m, r8_ref[...]) * _dot(sh2, s5_ref[...])
    o0 = jnp.sum(e0.reshape(K, 2 * PAD, M0), axis=0) * INVK
    o1 = jnp.sum(e1.reshape(K, 2 * PAD, 3 * M1), axis=0) * INVK
    o2 = jnp.sum(e2.reshape(K, 2 * PAD, 5 * M2), axis=0) * INVK
    out_ref[...] = jnp.concatenate([o0, o1, o2], axis=1)


def kernel(features, coord, mask, lin_w0, lin_w1, lin_w2, mlp_w1, mlp_b1, mlp_w2, mlp_b2):
    f32 = jnp.float32
    features = features.astype(f32)
    coord = coord.astype(f32)
    a_w = mlp_w1[:M0, :]
    b_w = mlp_w1[M0:M0 + BINS, :]
    c_w = mlp_w1[M0 + BINS:, :]
    b1 = mlp_b1.reshape(1, BINS)
    b2 = mlp_b2.reshape(1, M0 + M1 + M2)
    rpad = ((PAD, PAD), (0, 0))
    fpad = jnp.pad(features, rpad)
    cop = jnp.pad(coord, rpad)
    cot = coord.T

    wcol = lambda shp: pl.BlockSpec(shp, lambda i: (0, 0))
    main = pl.pallas_call(
        _band_kernel,
        grid=(NB,),
        in_specs=[
            wcol((NP2, D)), wcol((NP2, 3)),
            wcol((M0, D + 1)), wcol((M1, D + 1)), wcol((M2, D + 1)),
            wcol((BINS, M0)), wcol((BINS, D)),
            wcol((BINS, BINS)), wcol((BINS, 1)),
            wcol((M0 + M1 + M2, BINS)), wcol((M0 + M1 + M2, 1)),
            wcol((3 * M1, M1)), wcol((3 * M1, 3)),
            wcol((5 * M2, M2)), wcol((5 * M2, 5)),
        ],
        out_specs=pl.BlockSpec((RB, MOUT), lambda i: (i, 0)),
        out_shape=jax.ShapeDtypeStruct((N, MOUT), f32),
    )(fpad, cop,
      lin_w0.T, lin_w1.T, lin_w2.T, a_w.T, c_w.T,
      b_w.T, mlp_b1.reshape(BINS, 1), mlp_w2.T, mlp_b2.reshape(M0 + M1 + M2, 1),
      _R16.T, _S3M.T, _R8.T, _S5M.T)

    fix = pl.pallas_call(
        _boundary_kernel,
        out_shape=jax.ShapeDtypeStruct((2 * PAD, MOUT), f32),
    )(cot, coord, features,
      lin_w0, lin_w1, lin_w2, a_w, c_w,
      b_w, b1, mlp_w2, b2, _R16, _S3M, _R8, _S5M)

    return jnp.concatenate([fix[:PAD], main[PAD:N - PAD], fix[PAD:]], axis=0)


# speedup vs baseline: 68.8059x; 1.0638x over previous
import numpy as np
import jax
import jax.numpy as jnp
from jax.experimental import pallas as pl
from jax.experimental.pallas import tpu as pltpu

N = 4096
D = 128
K = 17
BINS = 32
M0, M1, M2 = 64, 16, 8
RB = 512
NB = N // RB
PAD = 8
NP2 = N + 2 * PAD
STEP = np.float32(4.0 / (BINS - 1))
INV_STEP = np.float32((BINS - 1) / 4.0)
S3 = np.float32(np.sqrt(3.0))
S15 = np.float32(np.sqrt(15.0))
HS15 = np.float32(np.sqrt(15.0) / 2.0)
HS5 = np.float32(np.sqrt(5.0) / 2.0)
INV112 = np.float32(1.0 / 1.12)
INVK = np.float32(1.0 / K)
BIG = np.float32(1e30)
MOUT = M0 + 3 * M1 + 5 * M2


def _expander(m, c):
    e = np.zeros((m, m * c), np.float32)
    for o in range(m):
        e[o, o * c:(o + 1) * c] = 1.0
    return e

def _tiler(c, m):
    e = np.zeros((c, m * c), np.float32)
    for o in range(m):
        for j in range(c):
            e[j, o * c + j] = 1.0
    return e

_R16 = _expander(M1, 3)
_S3M = _tiler(3, M1)
_R8 = _expander(M2, 5)
_S5M = _tiler(5, M2)


def _silu(x):
    return x * (1.0 / (1.0 + jnp.exp(-x)))


def _dot(a, b):
    return jnp.dot(a, b, preferred_element_type=jnp.float32)


def _band_kernel(fp_ref, cop_ref,
                 w0t_ref, w1t_ref, w2lt_ref, at_ref, ct_ref,
                 bt_ref, b1c_ref, w2t_ref, b2c_ref,
                 r16t_ref, s3t_ref, r8t_ref, s5t_ref,
                 out_ref):
    r0 = pl.program_id(0) * RB
    w = RB + 2 * PAD
    ftw = fp_ref[pl.ds(r0, w), :].T
    cow = cop_ref[pl.ds(r0, w), :].T
    t0w = _dot(w0t_ref[:, :D], ftw) + w0t_ref[:, D:D + 1]
    t1w = _dot(w1t_ref[:, :D], ftw) + w1t_ref[:, D:D + 1]
    t2w = _dot(w2lt_ref[:, :D], ftw) + w2lt_ref[:, D:D + 1]
    u0w = _dot(at_ref[...], t0w)
    ucb = _dot(ct_ref[...], ftw[:, PAD:PAD + RB])

    ccx = cow[0:1, PAD:PAD + RB]
    ccy = cow[1:2, PAD:PAD + RB]
    ccz = cow[2:3, PAD:PAD + RB]
    xs, ys, zs = [], [], []
    for t in range(K):
        xs.append(cow[0:1, t:t + RB] - ccx)
        ys.append(cow[1:2, t:t + RB] - ccy)
        zs.append(cow[2:3, t:t + RB] - ccz)
    x = jnp.concatenate(xs, 1)
    y = jnp.concatenate(ys, 1)
    z = jnp.concatenate(zs, 1)
    xx = x * x
    yy = y * y
    zz = z * z
    ns = xx + yy + zz
    norm = jnp.sqrt(jnp.where(ns == 0.0, 1.0, ns))
    valc = jax.lax.broadcasted_iota(jnp.int32, (BINS, 1), 0).astype(jnp.float32) * STEP
    dd = (norm - valc) * INV_STEP
    rad = jnp.exp(-(dd * dd)) * INV112
    sh1 = S3 * jnp.concatenate([x, y, z], 0)
    sh2 = jnp.concatenate([
        S15 * (x * y), S15 * (y * z), HS5 * (2.0 * zz - xx - yy),
        S15 * (x * z), HS15 * (xx - yy)], 0)

    t0c = jnp.concatenate([t0w[:, t:t + RB] for t in range(K)], 1)
    t1c = jnp.concatenate([t1w[:, t:t + RB] for t in range(K)], 1)
    t2c = jnp.concatenate([t2w[:, t:t + RB] for t in range(K)], 1)
    u0c = jnp.concatenate([u0w[:, t:t + RB] for t in range(K)], 1)
    ucc = jnp.concatenate([ucb] * K, 1)

    pre = u0c + ucc + b1c_ref[...] + _dot(bt_ref[...], rad)
    h = _silu(pre)
    mix = _dot(w2t_ref[...], h) + b2c_ref[...]
    e0 = t0c * mix[:M0, :]
    t1m = t1c * mix[M0:M0 + M1, :]
    e1 = _dot(r16t_ref[...], t1m) * _dot(s3t_ref[...], sh1)
    t2m = t2c * mix[M0 + M1:, :]
    e2 = _dot(r8t_ref[...], t2m) * _dot(s5t_ref[...], sh2)
    o0 = e0[:, 0:RB]
    o1 = e1[:, 0:RB]
    o2 = e2[:, 0:RB]
    for t in range(1, K):
        o0 = o0 + e0[:, t * RB:(t + 1) * RB]
        o1 = o1 + e1[:, t * RB:(t + 1) * RB]
        o2 = o2 + e2[:, t * RB:(t + 1) * RB]
    out_ref[...] = (jnp.concatenate([o0, o1, o2], 0) * INVK).T


def _geom_rows(vec):
    x = vec[:, 0:1]
    y = vec[:, 1:2]
    z = vec[:, 2:3]
    ns = x * x + y * y + z * z
    norm = jnp.sqrt(jnp.where(ns == 0.0, 1.0, ns))
    vals = jax.lax.broadcasted_iota(jnp.int32, (1, BINS), 1).astype(jnp.float32) * STEP
    dd = (norm - vals) * INV_STEP
    rad = jnp.exp(-(dd * dd)) * INV112
    sh1 = S3 * vec
    sh2 = jnp.concatenate([
        S15 * (x * y), S15 * (y * z), HS5 * (2.0 * z * z - x * x - y * y),
        S15 * (x * z), HS15 * (x * x - y * y)], axis=1)
    return rad, sh1, sh2


def _boundary_kernel(cot_ref, co_ref, f_ref,
                     w0_ref, w1_ref, w2l_ref, a_ref, c_ref,
                     b_ref, b1_ref, w2_ref, b2_ref, r16_ref, s3_ref, r8_ref, s5_ref,
                     out_ref):
    bco = jnp.concatenate([co_ref[0:PAD, :], co_ref[N - PAD:N, :]], axis=0)
    d2 = jnp.zeros((2 * PAD, N), jnp.float32)
    for c in range(3):
        diff = cot_ref[c:c + 1, :] - bco[:, c:c + 1]
        d2 = d2 + diff * diff
    j2 = jax.lax.broadcasted_iota(jnp.int32, (2 * PAD, N), 1)
    r1 = jax.lax.broadcasted_iota(jnp.int32, (2 * PAD, 1), 0)
    low = jnp.where(r1 < PAD, r1 + PAD, -1)
    high = jnp.where(r1 < PAD, N + 1, (N - 24) + r1)
    d2m = jnp.where((j2 <= low) | (j2 >= high), BIG, d2)
    spats = []
    for _ in range(PAD):
        m = jnp.min(d2m, axis=1, keepdims=True)
        am = jnp.min(jnp.where(d2m == m, j2, N), axis=1, keepdims=True)
        spats.append(am)
        d2m = jnp.where(j2 == am, BIG, d2m)
    spat = jnp.concatenate(spats, axis=1)
    tt = jax.lax.broadcasted_iota(jnp.int32, (2 * PAD, K), 1)
    rr = jax.lax.broadcasted_iota(jnp.int32, (2 * PAD, K), 0)
    ig = jnp.where(rr < PAD, rr, (N - 2 * PAD) + rr)
    fixedcnt = jnp.where(rr < PAD, rr + 9, 24 - rr)
    base = jnp.where(rr < PAD, tt, ig - PAD + tt)
    s_idx = tt - fixedcnt
    gath = jnp.zeros((2 * PAD, K), jnp.int32)
    for s in range(PAD):
        gath = gath + jnp.where(s_idx == s, spat[:, s:s + 1], 0)
    nei = jnp.where(tt < fixedcnt, base, gath)
    ohs = []
    for t in range(K):
        ohs.append((j2 == nei[:, t:t + 1]).astype(jnp.float32))
    oh = jnp.concatenate(ohs, axis=0)
    gf = _dot(oh, f_ref[...])
    gco = _dot(oh, co_ref[...])
    g0 = _dot(gf, w0_ref[:D, :]) + w0_ref[D:D + 1, :]
    g1 = _dot(gf, w1_ref[:D, :]) + w1_ref[D:D + 1, :]
    g2 = _dot(gf, w2l_ref[:D, :]) + w2l_ref[D:D + 1, :]
    gu = _dot(g0, a_ref[...])
    fc = jnp.concatenate([f_ref[0:PAD, :], f_ref[N - PAD:N, :]], axis=0)
    ucb = _dot(fc, c_ref[...])
    vec = gco - jnp.concatenate([bco] * K, 0)
    rad, sh1, sh2 = _geom_rows(vec)
    pre = (gu + jnp.concatenate([ucb] * K, 0) + b1_ref[...] + _dot(rad, b_ref[...]))
    h = _silu(pre)
    mix = _dot(h, w2_ref[...]) + b2_ref[...]
    e0 = g0 * mix[:, :M0]
    t1m = g1 * mix[:, M0:M0 + M1]
    e1 = _dot(t1m, r16_ref[...]) * _dot(sh1, s3_ref[...])
    t2m = g2 * mix[:, M0 + M1:]
    e2 = _dot(t2m, r8_ref[...]) * _dot(sh2, s5_ref[...])
    o0 = jnp.sum(e0.reshape(K, 2 * PAD, M0), axis=0) * INVK
    o1 = jnp.sum(e1.reshape(K, 2 * PAD, 3 * M1), axis=0) * INVK
    o2 = jnp.sum(e2.reshape(K, 2 * PAD, 5 * M2), axis=0) * INVK
    out_ref[...] = jnp.concatenate([o0, o1, o2], axis=1)


def kernel(features, coord, mask, lin_w0, lin_w1, lin_w2, mlp_w1, mlp_b1, mlp_w2, mlp_b2):
    f32 = jnp.float32
    features = features.astype(f32)
    coord = coord.astype(f32)
    a_w = mlp_w1[:M0, :]
    b_w = mlp_w1[M0:M0 + BINS, :]
    c_w = mlp_w1[M0 + BINS:, :]
    b1 = mlp_b1.reshape(1, BINS)
    b2 = mlp_b2.reshape(1, M0 + M1 + M2)
    rpad = ((PAD, PAD), (0, 0))
    fpad = jnp.pad(features, rpad)
    cop = jnp.pad(coord, rpad)
    cot = coord.T

    wcol = lambda shp: pl.BlockSpec(shp, lambda i: (0, 0))
    main = pl.pallas_call(
        _band_kernel,
        grid=(NB,),
        in_specs=[
            wcol((NP2, D)), wcol((NP2, 3)),
            wcol((M0, D + 1)), wcol((M1, D + 1)), wcol((M2, D + 1)),
            wcol((BINS, M0)), wcol((BINS, D)),
            wcol((BINS, BINS)), wcol((BINS, 1)),
            wcol((M0 + M1 + M2, BINS)), wcol((M0 + M1 + M2, 1)),
            wcol((3 * M1, M1)), wcol((3 * M1, 3)),
            wcol((5 * M2, M2)), wcol((5 * M2, 5)),
        ],
        out_specs=pl.BlockSpec((RB, MOUT), lambda i: (i, 0)),
        out_shape=jax.ShapeDtypeStruct((N, MOUT), f32),
    )(fpad, cop,
      lin_w0.T, lin_w1.T, lin_w2.T, a_w.T, c_w.T,
      b_w.T, mlp_b1.reshape(BINS, 1), mlp_w2.T, mlp_b2.reshape(M0 + M1 + M2, 1),
      _R16.T, _S3M.T, _R8.T, _S5M.T)

    fix = pl.pallas_call(
        _boundary_kernel,
        out_shape=jax.ShapeDtypeStruct((2 * PAD, MOUT), f32),
    )(cot, coord, features,
      lin_w0, lin_w1, lin_w2, a_w, c_w,
      b_w, b1, mlp_w2, b2, _R16, _S3M, _R8, _S5M)

    return jnp.concatenate([fix[:PAD], main[PAD:N - PAD], fix[PAD:]], axis=0)


# speedup vs baseline: 71.0151x; 1.0321x over previous
import numpy as np
import jax
import jax.numpy as jnp
from jax.experimental import pallas as pl
from jax.experimental.pallas import tpu as pltpu

N = 4096
D = 128
K = 17
BINS = 32
M0, M1, M2 = 64, 16, 8
RB = 1024
NB = N // RB
PAD = 8
NP2 = N + 2 * PAD
STEP = np.float32(4.0 / (BINS - 1))
INV_STEP = np.float32((BINS - 1) / 4.0)
S3 = np.float32(np.sqrt(3.0))
S15 = np.float32(np.sqrt(15.0))
HS15 = np.float32(np.sqrt(15.0) / 2.0)
HS5 = np.float32(np.sqrt(5.0) / 2.0)
INV112 = np.float32(1.0 / 1.12)
INVK = np.float32(1.0 / K)
BIG = np.float32(1e30)
MOUT = M0 + 3 * M1 + 5 * M2


def _expander(m, c):
    e = np.zeros((m, m * c), np.float32)
    for o in range(m):
        e[o, o * c:(o + 1) * c] = 1.0
    return e

def _tiler(c, m):
    e = np.zeros((c, m * c), np.float32)
    for o in range(m):
        for j in range(c):
            e[j, o * c + j] = 1.0
    return e

_R16 = _expander(M1, 3)
_S3M = _tiler(3, M1)
_R8 = _expander(M2, 5)
_S5M = _tiler(5, M2)


def _silu(x):
    return x * (1.0 / (1.0 + jnp.exp(-x)))


def _dot(a, b):
    return jnp.dot(a, b, preferred_element_type=jnp.float32)


def _band_kernel(fp_ref, cop_ref,
                 w0t_ref, w1t_ref, w2lt_ref, at_ref, ct_ref,
                 bt_ref, b1c_ref, w2t_ref, b2c_ref,
                 r16t_ref, s3t_ref, r8t_ref, s5t_ref,
                 out_ref):
    r0 = pl.program_id(0) * RB
    w = RB + 2 * PAD
    ftw = fp_ref[pl.ds(r0, w), :].T
    cow = cop_ref[pl.ds(r0, w), :].T
    t0w = _dot(w0t_ref[:, :D], ftw) + w0t_ref[:, D:D + 1]
    t1w = _dot(w1t_ref[:, :D], ftw) + w1t_ref[:, D:D + 1]
    t2w = _dot(w2lt_ref[:, :D], ftw) + w2lt_ref[:, D:D + 1]
    u0w = _dot(at_ref[...], t0w)
    ucb = _dot(ct_ref[...], ftw[:, PAD:PAD + RB])

    ccx = cow[0:1, PAD:PAD + RB]
    ccy = cow[1:2, PAD:PAD + RB]
    ccz = cow[2:3, PAD:PAD + RB]
    xs, ys, zs = [], [], []
    for t in range(K):
        xs.append(cow[0:1, t:t + RB] - ccx)
        ys.append(cow[1:2, t:t + RB] - ccy)
        zs.append(cow[2:3, t:t + RB] - ccz)
    x = jnp.concatenate(xs, 1)
    y = jnp.concatenate(ys, 1)
    z = jnp.concatenate(zs, 1)
    xx = x * x
    yy = y * y
    zz = z * z
    ns = xx + yy + zz
    norm = jnp.sqrt(jnp.where(ns == 0.0, 1.0, ns))
    valc = jax.lax.broadcasted_iota(jnp.int32, (BINS, 1), 0).astype(jnp.float32) * STEP
    dd = (norm - valc) * INV_STEP
    rad = jnp.exp(-(dd * dd)) * INV112
    sh1 = S3 * jnp.concatenate([x, y, z], 0)
    sh2 = jnp.concatenate([
        S15 * (x * y), S15 * (y * z), HS5 * (2.0 * zz - xx - yy),
        S15 * (x * z), HS15 * (xx - yy)], 0)

    t0c = jnp.concatenate([t0w[:, t:t + RB] for t in range(K)], 1)
    t1c = jnp.concatenate([t1w[:, t:t + RB] for t in range(K)], 1)
    t2c = jnp.concatenate([t2w[:, t:t + RB] for t in range(K)], 1)
    u0c = jnp.concatenate([u0w[:, t:t + RB] for t in range(K)], 1)
    ucc = jnp.concatenate([ucb] * K, 1)

    pre = u0c + ucc + b1c_ref[...] + _dot(bt_ref[...], rad)
    h = _silu(pre)
    mix = _dot(w2t_ref[...], h) + b2c_ref[...]
    e0 = t0c * mix[:M0, :]
    t1m = t1c * mix[M0:M0 + M1, :]
    e1 = _dot(r16t_ref[...], t1m) * _dot(s3t_ref[...], sh1)
    t2m = t2c * mix[M0 + M1:, :]
    e2 = _dot(r8t_ref[...], t2m) * _dot(s5t_ref[...], sh2)
    o0 = e0[:, 0:RB]
    o1 = e1[:, 0:RB]
    o2 = e2[:, 0:RB]
    for t in range(1, K):
        o0 = o0 + e0[:, t * RB:(t + 1) * RB]
        o1 = o1 + e1[:, t * RB:(t + 1) * RB]
        o2 = o2 + e2[:, t * RB:(t + 1) * RB]
    out_ref[...] = (jnp.concatenate([o0, o1, o2], 0) * INVK).T


def _geom_rows(vec):
    x = vec[:, 0:1]
    y = vec[:, 1:2]
    z = vec[:, 2:3]
    ns = x * x + y * y + z * z
    norm = jnp.sqrt(jnp.where(ns == 0.0, 1.0, ns))
    vals = jax.lax.broadcasted_iota(jnp.int32, (1, BINS), 1).astype(jnp.float32) * STEP
    dd = (norm - vals) * INV_STEP
    rad = jnp.exp(-(dd * dd)) * INV112
    sh1 = S3 * vec
    sh2 = jnp.concatenate([
        S15 * (x * y), S15 * (y * z), HS5 * (2.0 * z * z - x * x - y * y),
        S15 * (x * z), HS15 * (x * x - y * y)], axis=1)
    return rad, sh1, sh2


def _boundary_kernel(cot_ref, co_ref, f_ref,
                     w0_ref, w1_ref, w2l_ref, a_ref, c_ref,
                     b_ref, b1_ref, w2_ref, b2_ref, r16_ref, s3_ref, r8_ref, s5_ref,
                     out_ref):
    bco = jnp.concatenate([co_ref[0:PAD, :], co_ref[N - PAD:N, :]], axis=0)
    d2 = jnp.zeros((2 * PAD, N), jnp.float32)
    for c in range(3):
        diff = cot_ref[c:c + 1, :] - bco[:, c:c + 1]
        d2 = d2 + diff * diff
    j2 = jax.lax.broadcasted_iota(jnp.int32, (2 * PAD, N), 1)
    r1 = jax.lax.broadcasted_iota(jnp.int32, (2 * PAD, 1), 0)
    low = jnp.where(r1 < PAD, r1 + PAD, -1)
    high = jnp.where(r1 < PAD, N + 1, (N - 24) + r1)
    d2m = jnp.where((j2 <= low) | (j2 >= high), BIG, d2)
    spats = []
    for _ in range(PAD):
        m = jnp.min(d2m, axis=1, keepdims=True)
        am = jnp.min(jnp.where(d2m == m, j2, N), axis=1, keepdims=True)
        spats.append(am)
        d2m = jnp.where(j2 == am, BIG, d2m)
    spat = jnp.concatenate(spats, axis=1)
    tt = jax.lax.broadcasted_iota(jnp.int32, (2 * PAD, K), 1)
    rr = jax.lax.broadcasted_iota(jnp.int32, (2 * PAD, K), 0)
    ig = jnp.where(rr < PAD, rr, (N - 2 * PAD) + rr)
    fixedcnt = jnp.where(rr < PAD, rr + 9, 24 - rr)
    base = jnp.where(rr < PAD, tt, ig - PAD + tt)
    s_idx = tt - fixedcnt
    gath = jnp.zeros((2 * PAD, K), jnp.int32)
    for s in range(PAD):
        gath = gath + jnp.where(s_idx == s, spat[:, s:s + 1], 0)
    nei = jnp.where(tt < fixedcnt, base, gath)
    ohs = []
    for t in range(K):
        ohs.append((j2 == nei[:, t:t + 1]).astype(jnp.float32))
    oh = jnp.concatenate(ohs, axis=0)
    gf = _dot(oh, f_ref[...])
    gco = _dot(oh, co_ref[...])
    g0 = _dot(gf, w0_ref[:D, :]) + w0_ref[D:D + 1, :]
    g1 = _dot(gf, w1_ref[:D, :]) + w1_ref[D:D + 1, :]
    g2 = _dot(gf, w2l_ref[:D, :]) + w2l_ref[D:D + 1, :]
    gu = _dot(g0, a_ref[...])
    fc = jnp.concatenate([f_ref[0:PAD, :], f_ref[N - PAD:N, :]], axis=0)
    ucb = _dot(fc, c_ref[...])
    vec = gco - jnp.concatenate([bco] * K, 0)
    rad, sh1, sh2 = _geom_rows(vec)
    pre = (gu + jnp.concatenate([ucb] * K, 0) + b1_ref[...] + _dot(rad, b_ref[...]))
    h = _silu(pre)
    mix = _dot(h, w2_ref[...]) + b2_ref[...]
    e0 = g0 * mix[:, :M0]
    t1m = g1 * mix[:, M0:M0 + M1]
    e1 = _dot(t1m, r16_ref[...]) * _dot(sh1, s3_ref[...])
    t2m = g2 * mix[:, M0 + M1:]
    e2 = _dot(t2m, r8_ref[...]) * _dot(sh2, s5_ref[...])
    o0 = jnp.sum(e0.reshape(K, 2 * PAD, M0), axis=0) * INVK
    o1 = jnp.sum(e1.reshape(K, 2 * PAD, 3 * M1), axis=0) * INVK
    o2 = jnp.sum(e2.reshape(K, 2 * PAD, 5 * M2), axis=0) * INVK
    out_ref[...] = jnp.concatenate([o0, o1, o2], axis=1)


def kernel(features, coord, mask, lin_w0, lin_w1, lin_w2, mlp_w1, mlp_b1, mlp_w2, mlp_b2):
    f32 = jnp.float32
    features = features.astype(f32)
    coord = coord.astype(f32)
    a_w = mlp_w1[:M0, :]
    b_w = mlp_w1[M0:M0 + BINS, :]
    c_w = mlp_w1[M0 + BINS:, :]
    b1 = mlp_b1.reshape(1, BINS)
    b2 = mlp_b2.reshape(1, M0 + M1 + M2)
    rpad = ((PAD, PAD), (0, 0))
    fpad = jnp.pad(features, rpad)
    cop = jnp.pad(coord, rpad)
    cot = coord.T

    wcol = lambda shp: pl.BlockSpec(shp, lambda i: (0, 0))
    main = pl.pallas_call(
        _band_kernel,
        grid=(NB,),
        in_specs=[
            wcol((NP2, D)), wcol((NP2, 3)),
            wcol((M0, D + 1)), wcol((M1, D + 1)), wcol((M2, D + 1)),
            wcol((BINS, M0)), wcol((BINS, D)),
            wcol((BINS, BINS)), wcol((BINS, 1)),
            wcol((M0 + M1 + M2, BINS)), wcol((M0 + M1 + M2, 1)),
            wcol((3 * M1, M1)), wcol((3 * M1, 3)),
            wcol((5 * M2, M2)), wcol((5 * M2, 5)),
        ],
        out_specs=pl.BlockSpec((RB, MOUT), lambda i: (i, 0)),
        out_shape=jax.ShapeDtypeStruct((N, MOUT), f32),
    )(fpad, cop,
      lin_w0.T, lin_w1.T, lin_w2.T, a_w.T, c_w.T,
      b_w.T, mlp_b1.reshape(BINS, 1), mlp_w2.T, mlp_b2.reshape(M0 + M1 + M2, 1),
      _R16.T, _S3M.T, _R8.T, _S5M.T)

    fix = pl.pallas_call(
        _boundary_kernel,
        out_shape=jax.ShapeDtypeStruct((2 * PAD, MOUT), f32),
    )(cot, coord, features,
      lin_w0, lin_w1, lin_w2, a_w, c_w,
      b_w, b1, mlp_w2, b2, _R16, _S3M, _R8, _S5M)

    return jnp.concatenate([fix[:PAD], main[PAD:N - PAD], fix[PAD:]], axis=0)


# speedup vs baseline: 72.3460x; 1.0187x over previous
import numpy as np
import jax
import jax.numpy as jnp
from jax.experimental import pallas as pl
from jax.experimental.pallas import tpu as pltpu

N = 4096
D = 128
K = 17
BINS = 32
M0, M1, M2 = 64, 16, 8
RB = 1024
NB = N // RB
PAD = 8
NP2 = N + 2 * PAD
STEP = np.float32(4.0 / (BINS - 1))
INV_STEP = np.float32((BINS - 1) / 4.0)
S3 = np.float32(np.sqrt(3.0))
S15 = np.float32(np.sqrt(15.0))
HS15 = np.float32(np.sqrt(15.0) / 2.0)
HS5 = np.float32(np.sqrt(5.0) / 2.0)
INV112 = np.float32(1.0 / 1.12)
INVK = np.float32(1.0 / K)
BIG = np.float32(1e30)
MOUT = M0 + 3 * M1 + 5 * M2


def _expander(m, c):
    e = np.zeros((m, m * c), np.float32)
    for o in range(m):
        e[o, o * c:(o + 1) * c] = 1.0
    return e

def _tiler(c, m):
    e = np.zeros((c, m * c), np.float32)
    for o in range(m):
        for j in range(c):
            e[j, o * c + j] = 1.0
    return e

_R16 = _expander(M1, 3)
_S3M = _tiler(3, M1)
_R8 = _expander(M2, 5)
_S5M = _tiler(5, M2)


def _silu(x):
    return x * (1.0 / (1.0 + jnp.exp(-x)))


def _dot(a, b):
    return jnp.dot(a, b, preferred_element_type=jnp.float32)


def _band_step(pid, fp_ref, cop_ref, w0t_ref, w1t_ref, w2lt_ref, at_ref, ct_ref,
               bt_ref, b1c_ref, w2t_ref, b2c_ref,
               r16t_ref, s3t_ref, r8t_ref, s5t_ref, out_ref):
    r0 = pid * RB
    w = RB + 2 * PAD
    ftw = fp_ref[pl.ds(r0, w), :].T
    cow = cop_ref[pl.ds(r0, w), :].T
    t0w = _dot(w0t_ref[:, :D], ftw) + w0t_ref[:, D:D + 1]
    t1w = _dot(w1t_ref[:, :D], ftw) + w1t_ref[:, D:D + 1]
    t2w = _dot(w2lt_ref[:, :D], ftw) + w2lt_ref[:, D:D + 1]
    u0w = _dot(at_ref[...], t0w)
    ucb = _dot(ct_ref[...], ftw[:, PAD:PAD + RB])

    ccx = cow[0:1, PAD:PAD + RB]
    ccy = cow[1:2, PAD:PAD + RB]
    ccz = cow[2:3, PAD:PAD + RB]
    xs, ys, zs = [], [], []
    for t in range(K):
        xs.append(cow[0:1, t:t + RB] - ccx)
        ys.append(cow[1:2, t:t + RB] - ccy)
        zs.append(cow[2:3, t:t + RB] - ccz)
    x = jnp.concatenate(xs, 1)
    y = jnp.concatenate(ys, 1)
    z = jnp.concatenate(zs, 1)
    xx = x * x
    yy = y * y
    zz = z * z
    ns = xx + yy + zz
    norm = jnp.sqrt(jnp.where(ns == 0.0, 1.0, ns))
    valc = jax.lax.broadcasted_iota(jnp.int32, (BINS, 1), 0).astype(jnp.float32) * STEP
    dd = (norm - valc) * INV_STEP
    rad = jnp.exp(-(dd * dd)) * INV112
    sh1 = S3 * jnp.concatenate([x, y, z], 0)
    sh2 = jnp.concatenate([
        S15 * (x * y), S15 * (y * z), HS5 * (2.0 * zz - xx - yy),
        S15 * (x * z), HS15 * (xx - yy)], 0)

    t0c = jnp.concatenate([t0w[:, t:t + RB] for t in range(K)], 1)
    t1c = jnp.concatenate([t1w[:, t:t + RB] for t in range(K)], 1)
    t2c = jnp.concatenate([t2w[:, t:t + RB] for t in range(K)], 1)
    u0c = jnp.concatenate([u0w[:, t:t + RB] for t in range(K)], 1)
    ucc = jnp.concatenate([ucb] * K, 1)

    pre = u0c + ucc + b1c_ref[...] + _dot(bt_ref[...], rad)
    h = _silu(pre)
    mix = _dot(w2t_ref[...], h) + b2c_ref[...]
    e0 = t0c * mix[:M0, :]
    t1m = t1c * mix[M0:M0 + M1, :]
    e1 = _dot(r16t_ref[...], t1m) * _dot(s3t_ref[...], sh1)
    t2m = t2c * mix[M0 + M1:, :]
    e2 = _dot(r8t_ref[...], t2m) * _dot(s5t_ref[...], sh2)
    o0 = e0[:, 0:RB]
    o1 = e1[:, 0:RB]
    o2 = e2[:, 0:RB]
    for t in range(1, K):
        o0 = o0 + e0[:, t * RB:(t + 1) * RB]
        o1 = o1 + e1[:, t * RB:(t + 1) * RB]
        o2 = o2 + e2[:, t * RB:(t + 1) * RB]
    out_ref[pl.ds(r0, RB), :] = (jnp.concatenate([o0, o1, o2], 0) * INVK).T


def _geom_rows(vec):
    x = vec[:, 0:1]
    y = vec[:, 1:2]
    z = vec[:, 2:3]
    ns = x * x + y * y + z * z
    norm = jnp.sqrt(jnp.where(ns == 0.0, 1.0, ns))
    vals = jax.lax.broadcasted_iota(jnp.int32, (1, BINS), 1).astype(jnp.float32) * STEP
    dd = (norm - vals) * INV_STEP
    rad = jnp.exp(-(dd * dd)) * INV112
    sh1 = S3 * vec
    sh2 = jnp.concatenate([
        S15 * (x * y), S15 * (y * z), HS5 * (2.0 * z * z - x * x - y * y),
        S15 * (x * z), HS15 * (x * x - y * y)], axis=1)
    return rad, sh1, sh2


def _boundary_step(co_ref, f_ref, w0_ref, w1_ref, w2l_ref, a_ref, c_ref,
                   b_ref, b1_ref, w2_ref, b2_ref, r16_ref, s3_ref, r8_ref, s5_ref,
                   out_ref):
    cot = co_ref[...].T
    bco = jnp.concatenate([co_ref[0:PAD, :], co_ref[N - PAD:N, :]], axis=0)
    d2 = jnp.zeros((2 * PAD, N), jnp.float32)
    for c in range(3):
        diff = cot[c:c + 1, :] - bco[:, c:c + 1]
        d2 = d2 + diff * diff
    j2 = jax.lax.broadcasted_iota(jnp.int32, (2 * PAD, N), 1)
    r1 = jax.lax.broadcasted_iota(jnp.int32, (2 * PAD, 1), 0)
    low = jnp.where(r1 < PAD, r1 + PAD, -1)
    high = jnp.where(r1 < PAD, N + 1, (N - 24) + r1)
    d2m = jnp.where((j2 <= low) | (j2 >= high), BIG, d2)
    spats = []
    for _ in range(PAD):
        m = jnp.min(d2m, axis=1, keepdims=True)
        am = jnp.min(jnp.where(d2m == m, j2, N), axis=1, keepdims=True)
        spats.append(am)
        d2m = jnp.where(j2 == am, BIG, d2m)
    spat = jnp.concatenate(spats, axis=1)
    tt = jax.lax.broadcasted_iota(jnp.int32, (2 * PAD, K), 1)
    rr = jax.lax.broadcasted_iota(jnp.int32, (2 * PAD, K), 0)
    ig = jnp.where(rr < PAD, rr, (N - 2 * PAD) + rr)
    fixedcnt = jnp.where(rr < PAD, rr + 9, 24 - rr)
    base = jnp.where(rr < PAD, tt, ig - PAD + tt)
    s_idx = tt - fixedcnt
    gath = jnp.zeros((2 * PAD, K), jnp.int32)
    for s in range(PAD):
        gath = gath + jnp.where(s_idx == s, spat[:, s:s + 1], 0)
    nei = jnp.where(tt < fixedcnt, base, gath)
    ohs = []
    for t in range(K):
        ohs.append((j2 == nei[:, t:t + 1]).astype(jnp.float32))
    oh = jnp.concatenate(ohs, axis=0)
    gf = _dot(oh, f_ref[...])
    gco = _dot(oh, co_ref[...])
    g0 = _dot(gf, w0_ref[:D, :]) + w0_ref[D:D + 1, :]
    g1 = _dot(gf, w1_ref[:D, :]) + w1_ref[D:D + 1, :]
    g2 = _dot(gf, w2l_ref[:D, :]) + w2l_ref[D:D + 1, :]
    gu = _dot(g0, a_ref[...])
    fc = jnp.concatenate([f_ref[0:PAD, :], f_ref[N - PAD:N, :]], axis=0)
    ucb = _dot(fc, c_ref[...])
    vec = gco - jnp.concatenate([bco] * K, 0)
    rad, sh1, sh2 = _geom_rows(vec)
    pre = (gu + jnp.concatenate([ucb] * K, 0) + b1_ref[...] + _dot(rad, b_ref[...]))
    h = _silu(pre)
    mix = _dot(h, w2_ref[...]) + b2_ref[...]
    e0 = g0 * mix[:, :M0]
    t1m = g1 * mix[:, M0:M0 + M1]
    e1 = _dot(t1m, r16_ref[...]) * _dot(sh1, s3_ref[...])
    t2m = g2 * mix[:, M0 + M1:]
    e2 = _dot(t2m, r8_ref[...]) * _dot(sh2, s5_ref[...])
    o0 = jnp.sum(e0.reshape(K, 2 * PAD, M0), axis=0) * INVK
    o1 = jnp.sum(e1.reshape(K, 2 * PAD, 3 * M1), axis=0) * INVK
    o2 = jnp.sum(e2.reshape(K, 2 * PAD, 5 * M2), axis=0) * INVK
    res = jnp.concatenate([o0, o1, o2], axis=1)
    out_ref[0:PAD, :] = res[0:PAD, :]
    out_ref[N - PAD:N, :] = res[PAD:2 * PAD, :]


def _fused_kernel(fp_ref, cop_ref, f_ref, co_ref,
                  w0t_ref, w1t_ref, w2lt_ref, at_ref, ct_ref,
                  bt_ref, b1c_ref, w2t_ref, b2c_ref,
                  r16t_ref, s3t_ref, r8t_ref, s5t_ref,
                  w0_ref, w1_ref, w2l_ref, a_ref, c_ref,
                  b_ref, b1_ref, w2_ref, b2_ref,
                  r16_ref, s3_ref, r8_ref, s5_ref,
                  out_ref):
    pid = pl.program_id(0)

    @pl.when(pid < NB)
    def _band():
        _band_step(pid, fp_ref, cop_ref, w0t_ref, w1t_ref, w2lt_ref, at_ref,
                   ct_ref, bt_ref, b1c_ref, w2t_ref, b2c_ref,
                   r16t_ref, s3t_ref, r8t_ref, s5t_ref, out_ref)

    @pl.when(pid == NB)
    def _bnd():
        _boundary_step(co_ref, f_ref, w0_ref, w1_ref, w2l_ref, a_ref, c_ref,
                       b_ref, b1_ref, w2_ref, b2_ref,
                       r16_ref, s3_ref, r8_ref, s5_ref, out_ref)


def kernel(features, coord, mask, lin_w0, lin_w1, lin_w2, mlp_w1, mlp_b1, mlp_w2, mlp_b2):
    f32 = jnp.float32
    features = features.astype(f32)
    coord = coord.astype(f32)
    a_w = mlp_w1[:M0, :]
    b_w = mlp_w1[M0:M0 + BINS, :]
    c_w = mlp_w1[M0 + BINS:, :]
    b1 = mlp_b1.reshape(1, BINS)
    b2 = mlp_b2.reshape(1, M0 + M1 + M2)
    rpad = ((PAD, PAD), (0, 0))
    fpad = jnp.pad(features, rpad)
    cop = jnp.pad(coord, rpad)

    wcol = lambda shp: pl.BlockSpec(shp, lambda i: tuple(0 for _ in shp))
    out = pl.pallas_call(
        _fused_kernel,
        grid=(NB + 1,),
        in_specs=[
            wcol((NP2, D)), wcol((NP2, 3)), wcol((N, D)), wcol((N, 3)),
            wcol((M0, D + 1)), wcol((M1, D + 1)), wcol((M2, D + 1)),
            wcol((BINS, M0)), wcol((BINS, D)),
            wcol((BINS, BINS)), wcol((BINS, 1)),
            wcol((M0 + M1 + M2, BINS)), wcol((M0 + M1 + M2, 1)),
            wcol((3 * M1, M1)), wcol((3 * M1, 3)),
            wcol((5 * M2, M2)), wcol((5 * M2, 5)),
            wcol((D + 1, M0)), wcol((D + 1, M1)), wcol((D + 1, M2)),
            wcol((M0, BINS)), wcol((D, BINS)),
            wcol((BINS, BINS)), wcol((1, BINS)),
            wcol((BINS, M0 + M1 + M2)), wcol((1, M0 + M1 + M2)),
            wcol((M1, 3 * M1)), wcol((3, 3 * M1)),
            wcol((M2, 5 * M2)), wcol((5, 5 * M2)),
        ],
        out_specs=wcol((N, MOUT)),
        out_shape=jax.ShapeDtypeStruct((N, MOUT), f32),
    )(fpad, cop, features, coord,
      lin_w0.T, lin_w1.T, lin_w2.T, a_w.T, c_w.T,
      b_w.T, mlp_b1.reshape(BINS, 1), mlp_w2.T, mlp_b2.reshape(M0 + M1 + M2, 1),
      _R16.T, _S3M.T, _R8.T, _S5M.T,
      lin_w0, lin_w1, lin_w2, a_w, c_w,
      b_w, b1, mlp_w2, b2,
      _R16, _S3M, _R8, _S5M)
    return out
